# baseline (speedup 1.0000x reference)
import functools

import jax
import jax.numpy as jnp
from jax import lax
from jax.experimental import pallas as pl
from jax.experimental.pallas import tpu as pltpu
from jax.experimental.pallas import tpu_sc as plsc

N_NODES = 10000
N_EDGES = 320000
IN_DIM = 128
HID = 64
OUT_DIM = 2
NUM_GRAPHS = 64
EPS = 1e-5

NC = 2
NS = 16
NW = NC * NS
C = 125
EDGES_PER_W = N_EDGES // NW
CHUNKS = EDGES_PER_W // C
NPAD = 10000
ROWS_PER_SUB = NPAD // NS
ZROWS = 125
NBUF = 5
DBUF = 4

_HIGHEST = lax.Precision.DEFAULT


@functools.cache
def _mesh():
    return plsc.VectorSubcoreMesh(
        core_axis_name="c", subcore_axis_name="s", num_cores=NC, num_subcores=NS
    )


def _zero_fill(buf, ncols):
    zv = jnp.zeros((16,), jnp.float32)

    @pl.loop(0, ZROWS)
    def _(r):
        for cc in range(ncols // 16):
            buf[r, pl.ds(cc * 16, 16)] = zv


@functools.cache
def _sc_deg_kernel():
    return pl.kernel(
        _sc_deg_body,
        out_type=jax.ShapeDtypeStruct((NC, NPAD, 16), jnp.float32),
        mesh=_mesh(),
        compiler_params=pltpu.CompilerParams(use_tc_tiling_on_sc=False),
        scratch_types=[
            pltpu.VMEM((CHUNKS, C), jnp.int32),
            pltpu.VMEM((C, 16), jnp.float32),
            pltpu.VMEM((ZROWS, 16), jnp.float32),
            pltpu.VMEM_SHARED((NPAD, 16), jnp.float32),
            pltpu.SemaphoreType.DMA((DBUF,)),
        ],
    )


def _sc_deg_body(d_hbm, out_hbm, didx, ones_v, zbuf, acc, ssem):
    cid = lax.axis_index("c")
    sid = lax.axis_index("s")
    wid = sid * NC + cid

    _zero_fill(zbuf, 16)
    ov = jnp.ones((16,), jnp.float32)

    @pl.loop(0, C)
    def _(r):
        ones_v[r, pl.ds(0, 16)] = ov

    @pl.loop(0, ROWS_PER_SUB // ZROWS)
    def _(b):
        pltpu.sync_copy(zbuf, acc.at[pl.ds(sid * ROWS_PER_SUB + b * ZROWS, ZROWS)])

    plsc.subcore_barrier()

    pltpu.sync_copy(d_hbm.at[pl.ds(wid * CHUNKS, CHUNKS)], didx)

    for b in range(DBUF):
        pltpu.async_copy(ones_v, acc.at[didx.at[b]], ssem.at[b], add=True)

    @pl.loop(0, CHUNKS // DBUF)
    def _(t):
        j = t * DBUF
        for b in range(DBUF):
            pltpu.make_async_copy(ones_v, acc.at[didx.at[j + b]], ssem.at[b]).wait()

            @pl.when(j + DBUF + b < CHUNKS)
            def _():
                pltpu.async_copy(ones_v, acc.at[didx.at[j + DBUF + b]], ssem.at[b], add=True)

    plsc.subcore_barrier()
    base = sid * ROWS_PER_SUB
    pltpu.sync_copy(
        acc.at[pl.ds(base, ROWS_PER_SUB)],
        out_hbm.at[cid, pl.ds(base, ROWS_PER_SUB)],
    )


@functools.cache
def _sc_conv_kernel():
    return pl.kernel(
        _sc_conv_body,
        out_type=jax.ShapeDtypeStruct((NC, NPAD, HID), jnp.float32),
        mesh=_mesh(),
        compiler_params=pltpu.CompilerParams(use_tc_tiling_on_sc=False),
        scratch_types=[
            pltpu.VMEM((CHUNKS, C), jnp.int32),
            pltpu.VMEM((CHUNKS, C), jnp.int32),
        ]
        + [pltpu.VMEM((C, HID), jnp.float32) for _ in range(NBUF)]
        + [
            pltpu.VMEM((ZROWS, HID), jnp.float32),
            pltpu.VMEM_SHARED((NPAD, HID), jnp.float32),
            pltpu.SemaphoreType.DMA((NBUF,)),
            pltpu.SemaphoreType.DMA((NBUF,)),
        ],
    )


def _sc_conv_body(h_hbm, s_hbm, d_hbm, out_hbm, sidx, didx, *rest):
    rows = rest[:NBUF]
    zbuf, acc, gsem, ssem = rest[NBUF:]
    cid = lax.axis_index("c")
    sid = lax.axis_index("s")
    wid = sid * NC + cid

    _zero_fill(zbuf, HID)

    @pl.loop(0, ROWS_PER_SUB // ZROWS)
    def _(b):
        pltpu.sync_copy(zbuf, acc.at[pl.ds(sid * ROWS_PER_SUB + b * ZROWS, ZROWS)])

    plsc.subcore_barrier()

    base = wid * CHUNKS
    pltpu.sync_copy(s_hbm.at[pl.ds(base, CHUNKS)], sidx)
    pltpu.sync_copy(d_hbm.at[pl.ds(base, CHUNKS)], didx)

    for b in range(NBUF):
        pltpu.async_copy(h_hbm.at[sidx.at[b]], rows[b], gsem.at[b])

    @pl.loop(0, CHUNKS // NBUF)
    def _(t):
        j = t * NBUF
        for b in range(NBUF):
            pltpu.make_async_copy(h_hbm.at[sidx.at[j + b]], rows[b], gsem.at[b]).wait()
            pltpu.async_copy(rows[b], acc.at[didx.at[j + b]], ssem.at[b], add=True)
        for b in range(NBUF):
            pltpu.make_async_copy(rows[b], acc.at[didx.at[j + b]], ssem.at[b]).wait()

            @pl.when(j + NBUF + b < CHUNKS)
            def _():
                pltpu.async_copy(h_hbm.at[sidx.at[j + NBUF + b]], rows[b], gsem.at[b])

    plsc.subcore_barrier()
    rbase = sid * ROWS_PER_SUB
    pltpu.sync_copy(
        acc.at[pl.ds(rbase, ROWS_PER_SUB)],
        out_hbm.at[cid, pl.ds(rbase, ROWS_PER_SUB)],
    )


def _dinv_from_degp(degp):
    deg = degp[0, :N_NODES, 0] + degp[1, :N_NODES, 0] + 1.0
    return (1.0 / jnp.sqrt(deg))[:, None]


def _tc_pre_body(x_ref, w_ref, degp_ref, out_ref):
    h = lax.dot_general(
        x_ref[...], w_ref[...], (((1,), (0,)), ((), ())),
        precision=_HIGHEST, preferred_element_type=jnp.float32,
    )
    out_ref[...] = h * _dinv_from_degp(degp_ref[...])


def _tc_mid_body(degp_ref, p_ref, hp_ref, b_ref, g_ref, be_ref, w_ref, out_ref):
    dinv = _dinv_from_degp(degp_ref[...])
    o = (p_ref[0, :N_NODES] + p_ref[1, :N_NODES] + hp_ref[...]) * dinv + b_ref[...]
    mean = jnp.mean(o, axis=0, keepdims=True)
    var = jnp.mean((o - mean) ** 2, axis=0, keepdims=True)
    h = (o - mean) / jnp.sqrt(var + EPS) * g_ref[...] + be_ref[...]
    h = jnp.maximum(h, 0.0)
    h2 = lax.dot_general(
        h, w_ref[...], (((1,), (0,)), ((), ())),
        precision=_HIGHEST, preferred_element_type=jnp.float32,
    )
    out_ref[...] = h2 * dinv


def _tc_post_body(degp_ref, p_ref, hp_ref, b_ref, g_ref, be_ref, batch_ref, wc_ref, bc_ref, out_ref):
    dinv = _dinv_from_degp(degp_ref[...])
    o = (p_ref[0, :N_NODES] + p_ref[1, :N_NODES] + hp_ref[...]) * dinv + b_ref[...]
    mean = jnp.mean(o, axis=0, keepdims=True)
    var = jnp.mean((o - mean) ** 2, axis=0, keepdims=True)
    h = (o - mean) / jnp.sqrt(var + EPS) * g_ref[...] + be_ref[...]
    h = jnp.maximum(h, 0.0)
    gids = lax.broadcasted_iota(jnp.int32, (1, NUM_GRAPHS), 1)
    onehot = (batch_ref[...] == gids).astype(jnp.float32)
    sums = lax.dot_general(
        onehot, h, (((0,), (0,)), ((), ())),
        precision=_HIGHEST, preferred_element_type=jnp.float32,
    )
    counts = jnp.sum(onehot, axis=0)[:, None]
    pooled = sums / jnp.maximum(counts, 1.0)
    out_ref[...] = lax.dot_general(
        pooled, wc_ref[...], (((1,), (0,)), ((), ())),
        precision=_HIGHEST, preferred_element_type=jnp.float32,
    ) + bc_ref[...]


_tc_pre = pl.pallas_call(
    _tc_pre_body, out_shape=jax.ShapeDtypeStruct((N_NODES, HID), jnp.float32)
)
_tc_mid = pl.pallas_call(
    _tc_mid_body, out_shape=jax.ShapeDtypeStruct((N_NODES, HID), jnp.float32)
)
_tc_post = pl.pallas_call(
    _tc_post_body, out_shape=jax.ShapeDtypeStruct((NUM_GRAPHS, OUT_DIM), jnp.float32)
)


@jax.jit
def kernel(x, edge_index, batch, W1, b1, gamma1, beta1, W2, b2, gamma2, beta2, Wc, bc):
    src = edge_index[0].reshape(NW * CHUNKS, C)
    dst = edge_index[1].reshape(NW * CHUNKS, C)
    degp = _sc_deg_kernel()(dst)
    h1p = _tc_pre(x, W1, degp)
    p1 = _sc_conv_kernel()(h1p, src, dst)
    h2p = _tc_mid(degp, p1, h1p, b1[None, :], gamma1[None, :], beta1[None, :], W2)
    p2 = _sc_conv_kernel()(h2p, src, dst)
    return _tc_post(
        degp, p2, h2p, b2[None, :], gamma2[None, :], beta2[None, :],
        batch[:, None], Wc, bc[None, :],
    )


# speedup vs baseline: 1.0083x; 1.0083x over previous
import functools

import jax
import jax.numpy as jnp
from jax import lax
from jax.experimental import pallas as pl
from jax.experimental.pallas import tpu as pltpu
from jax.experimental.pallas import tpu_sc as plsc

N_NODES = 10000
N_EDGES = 320000
IN_DIM = 128
HID = 64
OUT_DIM = 2
NUM_GRAPHS = 64
EPS = 1e-5

NC = 2
NS = 16
NW = NC * NS
C = 125
EDGES_PER_W = N_EDGES // NW
CHUNKS = EDGES_PER_W // C
NPAD = 10000
ROWS_PER_SUB = NPAD // NS
ZROWS = 25
NBUF = 8
DBUF = 4

_HIGHEST = lax.Precision.DEFAULT


@functools.cache
def _mesh():
    return plsc.VectorSubcoreMesh(
        core_axis_name="c", subcore_axis_name="s", num_cores=NC, num_subcores=NS
    )


def _zero_fill(buf, ncols):
    zv = jnp.zeros((16,), jnp.float32)

    @pl.loop(0, ZROWS)
    def _(r):
        for cc in range(ncols // 16):
            buf[r, pl.ds(cc * 16, 16)] = zv


@functools.cache
def _sc_deg_kernel():
    return pl.kernel(
        _sc_deg_body,
        out_type=jax.ShapeDtypeStruct((NC, NPAD, 16), jnp.float32),
        mesh=_mesh(),
        compiler_params=pltpu.CompilerParams(use_tc_tiling_on_sc=False),
        scratch_types=[
            pltpu.VMEM((CHUNKS, C), jnp.int32),
            pltpu.VMEM((C, 16), jnp.float32),
            pltpu.VMEM((ZROWS, 16), jnp.float32),
            pltpu.VMEM_SHARED((NPAD, 16), jnp.float32),
            pltpu.SemaphoreType.DMA((DBUF,)),
        ],
    )


def _sc_deg_body(d_hbm, out_hbm, didx, ones_v, zbuf, acc, ssem):
    cid = lax.axis_index("c")
    sid = lax.axis_index("s")
    wid = sid * NC + cid

    _zero_fill(zbuf, 16)
    ov = jnp.ones((16,), jnp.float32)

    @pl.loop(0, C)
    def _(r):
        ones_v[r, pl.ds(0, 16)] = ov

    @pl.loop(0, ROWS_PER_SUB // ZROWS)
    def _(b):
        pltpu.sync_copy(zbuf, acc.at[pl.ds(sid * ROWS_PER_SUB + b * ZROWS, ZROWS)])

    plsc.subcore_barrier()

    pltpu.sync_copy(d_hbm.at[pl.ds(wid * CHUNKS, CHUNKS)], didx)

    for b in range(DBUF):
        pltpu.async_copy(ones_v, acc.at[didx.at[b]], ssem.at[b], add=True)

    @pl.loop(0, CHUNKS // DBUF)
    def _(t):
        j = t * DBUF
        for b in range(DBUF):
            pltpu.make_async_copy(ones_v, acc.at[didx.at[j + b]], ssem.at[b]).wait()

            @pl.when(j + DBUF + b < CHUNKS)
            def _():
                pltpu.async_copy(ones_v, acc.at[didx.at[j + DBUF + b]], ssem.at[b], add=True)

    plsc.subcore_barrier()
    base = sid * ROWS_PER_SUB
    pltpu.sync_copy(
        acc.at[pl.ds(base, ROWS_PER_SUB)],
        out_hbm.at[cid, pl.ds(base, ROWS_PER_SUB)],
    )


@functools.cache
def _sc_conv_kernel():
    return pl.kernel(
        _sc_conv_body,
        out_type=jax.ShapeDtypeStruct((NC, NPAD, HID), jnp.float32),
        mesh=_mesh(),
        compiler_params=pltpu.CompilerParams(use_tc_tiling_on_sc=False),
        scratch_types=[
            pltpu.VMEM((CHUNKS, C), jnp.int32),
            pltpu.VMEM((CHUNKS, C), jnp.int32),
        ]
        + [pltpu.VMEM((C, HID), jnp.float32) for _ in range(NBUF)]
        + [
            pltpu.VMEM((ZROWS, HID), jnp.float32),
            pltpu.VMEM_SHARED((NPAD, HID), jnp.float32),
        ]
        + [pltpu.SemaphoreType.DMA for _ in range(2 * NBUF)],
    )


def _sc_conv_body(h_hbm, s_hbm, d_hbm, out_hbm, sidx, didx, *rest):
    rows = rest[:NBUF]
    zbuf, acc = rest[NBUF], rest[NBUF + 1]
    gsem = rest[NBUF + 2:NBUF + 2 + NBUF]
    ssem = rest[NBUF + 2 + NBUF:]
    cid = lax.axis_index("c")
    sid = lax.axis_index("s")
    wid = sid * NC + cid

    _zero_fill(zbuf, HID)

    @pl.loop(0, ROWS_PER_SUB // ZROWS)
    def _(b):
        pltpu.sync_copy(zbuf, acc.at[pl.ds(sid * ROWS_PER_SUB + b * ZROWS, ZROWS)])

    plsc.subcore_barrier()

    base = wid * CHUNKS
    pltpu.sync_copy(s_hbm.at[pl.ds(base, CHUNKS)], sidx)
    pltpu.sync_copy(d_hbm.at[pl.ds(base, CHUNKS)], didx)

    for b in range(NBUF):
        pltpu.async_copy(h_hbm.at[sidx.at[b]], rows[b], gsem[b])

    @pl.loop(0, CHUNKS // NBUF)
    def _(t):
        j = t * NBUF
        for b in range(NBUF):
            pltpu.make_async_copy(h_hbm.at[sidx.at[j + b]], rows[b], gsem[b]).wait()
            pltpu.async_copy(rows[b], acc.at[didx.at[j + b]], ssem[b], add=True)
        for b in range(NBUF):
            pltpu.make_async_copy(rows[b], acc.at[didx.at[j + b]], ssem[b]).wait()

            @pl.when(j + NBUF + b < CHUNKS)
            def _():
                pltpu.async_copy(h_hbm.at[sidx.at[j + NBUF + b]], rows[b], gsem[b])

    plsc.subcore_barrier()
    rbase = sid * ROWS_PER_SUB
    pltpu.sync_copy(
        acc.at[pl.ds(rbase, ROWS_PER_SUB)],
        out_hbm.at[cid, pl.ds(rbase, ROWS_PER_SUB)],
    )


def _dinv_from_degp(degp):
    deg = degp[0, :N_NODES, 0] + degp[1, :N_NODES, 0] + 1.0
    return (1.0 / jnp.sqrt(deg))[:, None]


def _tc_pre_body(x_ref, w_ref, degp_ref, out_ref):
    h = lax.dot_general(
        x_ref[...], w_ref[...], (((1,), (0,)), ((), ())),
        precision=_HIGHEST, preferred_element_type=jnp.float32,
    )
    out_ref[...] = h * _dinv_from_degp(degp_ref[...])


def _tc_mid_body(degp_ref, p_ref, hp_ref, b_ref, g_ref, be_ref, w_ref, out_ref):
    dinv = _dinv_from_degp(degp_ref[...])
    o = (p_ref[0, :N_NODES] + p_ref[1, :N_NODES] + hp_ref[...]) * dinv + b_ref[...]
    mean = jnp.mean(o, axis=0, keepdims=True)
    var = jnp.mean((o - mean) ** 2, axis=0, keepdims=True)
    h = (o - mean) / jnp.sqrt(var + EPS) * g_ref[...] + be_ref[...]
    h = jnp.maximum(h, 0.0)
    h2 = lax.dot_general(
        h, w_ref[...], (((1,), (0,)), ((), ())),
        precision=_HIGHEST, preferred_element_type=jnp.float32,
    )
    out_ref[...] = h2 * dinv


def _tc_post_body(degp_ref, p_ref, hp_ref, b_ref, g_ref, be_ref, batch_ref, wc_ref, bc_ref, out_ref):
    dinv = _dinv_from_degp(degp_ref[...])
    o = (p_ref[0, :N_NODES] + p_ref[1, :N_NODES] + hp_ref[...]) * dinv + b_ref[...]
    mean = jnp.mean(o, axis=0, keepdims=True)
    var = jnp.mean((o - mean) ** 2, axis=0, keepdims=True)
    h = (o - mean) / jnp.sqrt(var + EPS) * g_ref[...] + be_ref[...]
    h = jnp.maximum(h, 0.0)
    gids = lax.broadcasted_iota(jnp.int32, (1, NUM_GRAPHS), 1)
    onehot = (batch_ref[...] == gids).astype(jnp.float32)
    sums = lax.dot_general(
        onehot, h, (((0,), (0,)), ((), ())),
        precision=_HIGHEST, preferred_element_type=jnp.float32,
    )
    counts = jnp.sum(onehot, axis=0)[:, None]
    pooled = sums / jnp.maximum(counts, 1.0)
    out_ref[...] = lax.dot_general(
        pooled, wc_ref[...], (((1,), (0,)), ((), ())),
        precision=_HIGHEST, preferred_element_type=jnp.float32,
    ) + bc_ref[...]


_tc_pre = pl.pallas_call(
    _tc_pre_body, out_shape=jax.ShapeDtypeStruct((N_NODES, HID), jnp.float32)
)
_tc_mid = pl.pallas_call(
    _tc_mid_body, out_shape=jax.ShapeDtypeStruct((N_NODES, HID), jnp.float32)
)
_tc_post = pl.pallas_call(
    _tc_post_body, out_shape=jax.ShapeDtypeStruct((NUM_GRAPHS, OUT_DIM), jnp.float32)
)


@jax.jit
def kernel(x, edge_index, batch, W1, b1, gamma1, beta1, W2, b2, gamma2, beta2, Wc, bc):
    src = edge_index[0].reshape(NW * CHUNKS, C)
    dst = edge_index[1].reshape(NW * CHUNKS, C)
    degp = _sc_deg_kernel()(dst)
    h1p = _tc_pre(x, W1, degp)
    p1 = _sc_conv_kernel()(h1p, src, dst)
    h2p = _tc_mid(degp, p1, h1p, b1[None, :], gamma1[None, :], beta1[None, :], W2)
    p2 = _sc_conv_kernel()(h2p, src, dst)
    return _tc_post(
        degp, p2, h2p, b2[None, :], gamma2[None, :], beta2[None, :],
        batch[:, None], Wc, bc[None, :],
    )


# speedup vs baseline: 1.0271x; 1.0186x over previous
import functools

import jax
import jax.numpy as jnp
from jax import lax
from jax.experimental import pallas as pl
from jax.experimental.pallas import tpu as pltpu
from jax.experimental.pallas import tpu_sc as plsc

N_NODES = 10000
N_EDGES = 320000
IN_DIM = 128
HID = 64
OUT_DIM = 2
NUM_GRAPHS = 64
EPS = 1e-5

NC = 2
NS = 16
NW = NC * NS
C = 125
EDGES_PER_W = N_EDGES // NW
CHUNKS = EDGES_PER_W // C
NPAD = 10000
ROWS_PER_SUB = NPAD // NS
ZROWS = 125
NBUF = 8
DBUF = 4

_HIGHEST = lax.Precision.DEFAULT


@functools.cache
def _mesh():
    return plsc.VectorSubcoreMesh(
        core_axis_name="c", subcore_axis_name="s", num_cores=NC, num_subcores=NS
    )


def _zero_fill(buf, ncols):
    zv = jnp.zeros((16,), jnp.float32)

    @pl.loop(0, ZROWS)
    def _(r):
        for cc in range(ncols // 16):
            buf[r, pl.ds(cc * 16, 16)] = zv


@functools.cache
def _sc_deg_kernel():
    return pl.kernel(
        _sc_deg_body,
        out_type=jax.ShapeDtypeStruct((NC, NPAD, 16), jnp.float32),
        mesh=_mesh(),
        compiler_params=pltpu.CompilerParams(use_tc_tiling_on_sc=False),
        scratch_types=[
            pltpu.VMEM((CHUNKS, C), jnp.int32),
            pltpu.VMEM((C, 16), jnp.float32),
            pltpu.VMEM((ZROWS, 16), jnp.float32),
            pltpu.VMEM_SHARED((NPAD, 16), jnp.float32),
            pltpu.SemaphoreType.DMA((DBUF,)),
        ],
    )


def _sc_deg_body(d_hbm, out_hbm, didx, ones_v, zbuf, acc, ssem):
    cid = lax.axis_index("c")
    sid = lax.axis_index("s")
    wid = sid * NC + cid

    _zero_fill(zbuf, 16)
    ov = jnp.ones((16,), jnp.float32)

    @pl.loop(0, C)
    def _(r):
        ones_v[r, pl.ds(0, 16)] = ov

    @pl.loop(0, ROWS_PER_SUB // ZROWS)
    def _(b):
        pltpu.sync_copy(zbuf, acc.at[pl.ds(sid * ROWS_PER_SUB + b * ZROWS, ZROWS)])

    plsc.subcore_barrier()

    pltpu.sync_copy(d_hbm.at[pl.ds(wid * CHUNKS, CHUNKS)], didx)

    for b in range(DBUF):
        pltpu.async_copy(ones_v, acc.at[didx.at[b]], ssem.at[b], add=True)

    @pl.loop(0, CHUNKS // DBUF)
    def _(t):
        j = t * DBUF
        for b in range(DBUF):
            pltpu.make_async_copy(ones_v, acc.at[didx.at[j + b]], ssem.at[b]).wait()

            @pl.when(j + DBUF + b < CHUNKS)
            def _():
                pltpu.async_copy(ones_v, acc.at[didx.at[j + DBUF + b]], ssem.at[b], add=True)

    plsc.subcore_barrier()
    base = sid * ROWS_PER_SUB
    pltpu.sync_copy(
        acc.at[pl.ds(base, ROWS_PER_SUB)],
        out_hbm.at[cid, pl.ds(base, ROWS_PER_SUB)],
    )


@functools.cache
def _sc_conv_kernel():
    return pl.kernel(
        _sc_conv_body,
        out_type=jax.ShapeDtypeStruct((NC, NPAD, HID), jnp.float32),
        mesh=_mesh(),
        compiler_params=pltpu.CompilerParams(use_tc_tiling_on_sc=False),
        scratch_types=[
            pltpu.VMEM((CHUNKS, C), jnp.int32),
            pltpu.VMEM((CHUNKS, C), jnp.int32),
        ]
        + [pltpu.VMEM((C, HID), jnp.float32) for _ in range(NBUF)]
        + [
            pltpu.VMEM_SHARED((NPAD, HID), jnp.float32),
        ]
        + [pltpu.SemaphoreType.DMA for _ in range(2 * NBUF)],
    )


def _sc_conv_body(h_hbm, s_hbm, d_hbm, out_hbm, sidx, didx, *rest):
    rows = rest[:NBUF]
    acc = rest[NBUF]
    gsem = rest[NBUF + 1:NBUF + 1 + NBUF]
    ssem = rest[NBUF + 1 + NBUF:]
    cid = lax.axis_index("c")
    sid = lax.axis_index("s")
    wid = sid * NC + cid

    _zero_fill(rows[0], HID)
    nz = ROWS_PER_SUB // ZROWS
    for z in range(nz):
        pltpu.async_copy(
            rows[0], acc.at[pl.ds(sid * ROWS_PER_SUB + z * ZROWS, ZROWS)], ssem[z]
        )

    base = wid * CHUNKS
    pltpu.sync_copy(s_hbm.at[pl.ds(base, CHUNKS)], sidx)
    pltpu.sync_copy(d_hbm.at[pl.ds(base, CHUNKS)], didx)

    for z in range(nz):
        pltpu.make_async_copy(
            rows[0], acc.at[pl.ds(sid * ROWS_PER_SUB + z * ZROWS, ZROWS)], ssem[z]
        ).wait()

    plsc.subcore_barrier()

    for b in range(NBUF):
        pltpu.async_copy(h_hbm.at[sidx.at[b]], rows[b], gsem[b])

    @pl.loop(0, CHUNKS // NBUF)
    def _(t):
        j = t * NBUF
        for b in range(NBUF):
            pltpu.make_async_copy(h_hbm.at[sidx.at[j + b]], rows[b], gsem[b]).wait()
            pltpu.async_copy(rows[b], acc.at[didx.at[j + b]], ssem[b], add=True)
        for b in range(NBUF):
            pltpu.make_async_copy(rows[b], acc.at[didx.at[j + b]], ssem[b]).wait()

            @pl.when(j + NBUF + b < CHUNKS)
            def _():
                pltpu.async_copy(h_hbm.at[sidx.at[j + NBUF + b]], rows[b], gsem[b])

    plsc.subcore_barrier()
    rbase = sid * ROWS_PER_SUB
    pltpu.sync_copy(
        acc.at[pl.ds(rbase, ROWS_PER_SUB)],
        out_hbm.at[cid, pl.ds(rbase, ROWS_PER_SUB)],
    )


def _dinv_from_degp(degp):
    deg = degp[0, :N_NODES, 0] + degp[1, :N_NODES, 0] + 1.0
    return (1.0 / jnp.sqrt(deg))[:, None]


def _tc_pre_body(x_ref, w_ref, degp_ref, out_ref):
    h = lax.dot_general(
        x_ref[...], w_ref[...], (((1,), (0,)), ((), ())),
        precision=_HIGHEST, preferred_element_type=jnp.float32,
    )
    out_ref[...] = h * _dinv_from_degp(degp_ref[...])


def _tc_mid_body(degp_ref, p_ref, hp_ref, b_ref, g_ref, be_ref, w_ref, out_ref):
    dinv = _dinv_from_degp(degp_ref[...])
    o = (p_ref[0, :N_NODES] + p_ref[1, :N_NODES] + hp_ref[...]) * dinv + b_ref[...]
    mean = jnp.mean(o, axis=0, keepdims=True)
    var = jnp.mean((o - mean) ** 2, axis=0, keepdims=True)
    h = (o - mean) / jnp.sqrt(var + EPS) * g_ref[...] + be_ref[...]
    h = jnp.maximum(h, 0.0)
    h2 = lax.dot_general(
        h, w_ref[...], (((1,), (0,)), ((), ())),
        precision=_HIGHEST, preferred_element_type=jnp.float32,
    )
    out_ref[...] = h2 * dinv


def _tc_post_body(degp_ref, p_ref, hp_ref, b_ref, g_ref, be_ref, batch_ref, wc_ref, bc_ref, out_ref):
    dinv = _dinv_from_degp(degp_ref[...])
    o = (p_ref[0, :N_NODES] + p_ref[1, :N_NODES] + hp_ref[...]) * dinv + b_ref[...]
    mean = jnp.mean(o, axis=0, keepdims=True)
    var = jnp.mean((o - mean) ** 2, axis=0, keepdims=True)
    h = (o - mean) / jnp.sqrt(var + EPS) * g_ref[...] + be_ref[...]
    h = jnp.maximum(h, 0.0)
    gids = lax.broadcasted_iota(jnp.int32, (1, NUM_GRAPHS), 1)
    onehot = (batch_ref[...] == gids).astype(jnp.float32)
    sums = lax.dot_general(
        onehot, h, (((0,), (0,)), ((), ())),
        precision=_HIGHEST, preferred_element_type=jnp.float32,
    )
    counts = jnp.sum(onehot, axis=0)[:, None]
    pooled = sums / jnp.maximum(counts, 1.0)
    out_ref[...] = lax.dot_general(
        pooled, wc_ref[...], (((1,), (0,)), ((), ())),
        precision=_HIGHEST, preferred_element_type=jnp.float32,
    ) + bc_ref[...]


_tc_pre = pl.pallas_call(
    _tc_pre_body, out_shape=jax.ShapeDtypeStruct((N_NODES, HID), jnp.float32)
)
_tc_mid = pl.pallas_call(
    _tc_mid_body, out_shape=jax.ShapeDtypeStruct((N_NODES, HID), jnp.float32)
)
_tc_post = pl.pallas_call(
    _tc_post_body, out_shape=jax.ShapeDtypeStruct((NUM_GRAPHS, OUT_DIM), jnp.float32)
)


@jax.jit
def kernel(x, edge_index, batch, W1, b1, gamma1, beta1, W2, b2, gamma2, beta2, Wc, bc):
    src = edge_index[0].reshape(NW * CHUNKS, C)
    dst = edge_index[1].reshape(NW * CHUNKS, C)
    degp = _sc_deg_kernel()(dst)
    h1p = _tc_pre(x, W1, degp)
    p1 = _sc_conv_kernel()(h1p, src, dst)
    h2p = _tc_mid(degp, p1, h1p, b1[None, :], gamma1[None, :], beta1[None, :], W2)
    p2 = _sc_conv_kernel()(h2p, src, dst)
    return _tc_post(
        degp, p2, h2p, b2[None, :], gamma2[None, :], beta2[None, :],
        batch[:, None], Wc, bc[None, :],
    )


# speedup vs baseline: 1.0286x; 1.0014x over previous
import functools

import jax
import jax.numpy as jnp
from jax import lax
from jax.experimental import pallas as pl
from jax.experimental.pallas import tpu as pltpu
from jax.experimental.pallas import tpu_sc as plsc

N_NODES = 10000
N_EDGES = 320000
IN_DIM = 128
HID = 64
OUT_DIM = 2
NUM_GRAPHS = 64
EPS = 1e-5

NC = 2
NS = 16
NW = NC * NS
C = 125
EDGES_PER_W = N_EDGES // NW
CHUNKS = EDGES_PER_W // C
NPAD = 10000
ROWS_PER_SUB = NPAD // NS
ZROWS = 125
NBUF = 8
DBUF = 5

_HIGHEST = lax.Precision.DEFAULT


@functools.cache
def _mesh():
    return plsc.VectorSubcoreMesh(
        core_axis_name="c", subcore_axis_name="s", num_cores=NC, num_subcores=NS
    )


def _zero_fill(buf, ncols):
    zv = jnp.zeros((16,), jnp.float32)

    @pl.loop(0, ZROWS)
    def _(r):
        for cc in range(ncols // 16):
            buf[r, pl.ds(cc * 16, 16)] = zv


@functools.cache
def _sc_deg_kernel():
    return pl.kernel(
        _sc_deg_body,
        out_type=jax.ShapeDtypeStruct((NC, NPAD, 16), jnp.float32),
        mesh=_mesh(),
        compiler_params=pltpu.CompilerParams(use_tc_tiling_on_sc=False),
        scratch_types=[
            pltpu.VMEM((CHUNKS, C), jnp.int32),
            pltpu.VMEM((C, 16), jnp.float32),
            pltpu.VMEM((ZROWS, 16), jnp.float32),
            pltpu.VMEM_SHARED((NPAD, 16), jnp.float32),
            pltpu.SemaphoreType.DMA((DBUF,)),
        ],
    )


def _sc_deg_body(d_hbm, out_hbm, didx, ones_v, zbuf, acc, ssem):
    cid = lax.axis_index("c")
    sid = lax.axis_index("s")
    wid = sid * NC + cid

    _zero_fill(zbuf, 16)
    nz = ROWS_PER_SUB // ZROWS
    for z in range(nz):
        pltpu.async_copy(
            zbuf, acc.at[pl.ds(sid * ROWS_PER_SUB + z * ZROWS, ZROWS)], ssem.at[z]
        )

    ov = jnp.ones((16,), jnp.float32)

    @pl.loop(0, C)
    def _(r):
        ones_v[r, pl.ds(0, 16)] = ov

    pltpu.sync_copy(d_hbm.at[pl.ds(wid * CHUNKS, CHUNKS)], didx)

    for z in range(nz):
        pltpu.make_async_copy(
            zbuf, acc.at[pl.ds(sid * ROWS_PER_SUB + z * ZROWS, ZROWS)], ssem.at[z]
        ).wait()

    plsc.subcore_barrier()

    for b in range(DBUF):
        pltpu.async_copy(ones_v, acc.at[didx.at[b]], ssem.at[b], add=True)

    @pl.loop(0, CHUNKS // DBUF)
    def _(t):
        j = t * DBUF
        for b in range(DBUF):
            pltpu.make_async_copy(ones_v, acc.at[didx.at[j + b]], ssem.at[b]).wait()

            @pl.when(j + DBUF + b < CHUNKS)
            def _():
                pltpu.async_copy(ones_v, acc.at[didx.at[j + DBUF + b]], ssem.at[b], add=True)

    plsc.subcore_barrier()
    base = sid * ROWS_PER_SUB
    pltpu.sync_copy(
        acc.at[pl.ds(base, ROWS_PER_SUB)],
        out_hbm.at[cid, pl.ds(base, ROWS_PER_SUB)],
    )


@functools.cache
def _sc_conv_kernel():
    return pl.kernel(
        _sc_conv_body,
        out_type=jax.ShapeDtypeStruct((NC, NPAD, HID), jnp.float32),
        mesh=_mesh(),
        compiler_params=pltpu.CompilerParams(use_tc_tiling_on_sc=False),
        scratch_types=[
            pltpu.VMEM((CHUNKS, C), jnp.int32),
            pltpu.VMEM((CHUNKS, C), jnp.int32),
        ]
        + [pltpu.VMEM((C, HID), jnp.float32) for _ in range(NBUF)]
        + [
            pltpu.VMEM_SHARED((NPAD, HID), jnp.float32),
        ]
        + [pltpu.SemaphoreType.DMA for _ in range(2 * NBUF)],
    )


def _sc_conv_body(h_hbm, s_hbm, d_hbm, out_hbm, sidx, didx, *rest):
    rows = rest[:NBUF]
    acc = rest[NBUF]
    gsem = rest[NBUF + 1:NBUF + 1 + NBUF]
    ssem = rest[NBUF + 1 + NBUF:]
    cid = lax.axis_index("c")
    sid = lax.axis_index("s")
    wid = sid * NC + cid

    _zero_fill(rows[0], HID)
    nz = ROWS_PER_SUB // ZROWS
    for z in range(nz):
        pltpu.async_copy(
            rows[0], acc.at[pl.ds(sid * ROWS_PER_SUB + z * ZROWS, ZROWS)], ssem[z]
        )

    base = wid * CHUNKS
    pltpu.sync_copy(s_hbm.at[pl.ds(base, CHUNKS)], sidx)
    pltpu.sync_copy(d_hbm.at[pl.ds(base, CHUNKS)], didx)

    for z in range(nz):
        pltpu.make_async_copy(
            rows[0], acc.at[pl.ds(sid * ROWS_PER_SUB + z * ZROWS, ZROWS)], ssem[z]
        ).wait()

    plsc.subcore_barrier()

    for b in range(NBUF):
        pltpu.async_copy(h_hbm.at[sidx.at[b]], rows[b], gsem[b])

    @pl.loop(0, CHUNKS // NBUF)
    def _(t):
        j = t * NBUF
        for b in range(NBUF):
            pltpu.make_async_copy(h_hbm.at[sidx.at[j + b]], rows[b], gsem[b]).wait()
            pltpu.async_copy(rows[b], acc.at[didx.at[j + b]], ssem[b], add=True)
        for b in range(NBUF):
            pltpu.make_async_copy(rows[b], acc.at[didx.at[j + b]], ssem[b]).wait()

            @pl.when(j + NBUF + b < CHUNKS)
            def _():
                pltpu.async_copy(h_hbm.at[sidx.at[j + NBUF + b]], rows[b], gsem[b])

    plsc.subcore_barrier()
    rbase = sid * ROWS_PER_SUB
    pltpu.sync_copy(
        acc.at[pl.ds(rbase, ROWS_PER_SUB)],
        out_hbm.at[cid, pl.ds(rbase, ROWS_PER_SUB)],
    )


def _dinv_from_degp(degp):
    deg = degp[0, :N_NODES, 0] + degp[1, :N_NODES, 0] + 1.0
    return (1.0 / jnp.sqrt(deg))[:, None]


def _tc_pre_body(x_ref, w_ref, degp_ref, out_ref):
    h = lax.dot_general(
        x_ref[...], w_ref[...], (((1,), (0,)), ((), ())),
        precision=_HIGHEST, preferred_element_type=jnp.float32,
    )
    out_ref[...] = h * _dinv_from_degp(degp_ref[...])


def _tc_mid_body(degp_ref, p_ref, hp_ref, b_ref, g_ref, be_ref, w_ref, out_ref):
    dinv = _dinv_from_degp(degp_ref[...])
    o = (p_ref[0, :N_NODES] + p_ref[1, :N_NODES] + hp_ref[...]) * dinv + b_ref[...]
    mean = jnp.mean(o, axis=0, keepdims=True)
    var = jnp.mean((o - mean) ** 2, axis=0, keepdims=True)
    h = (o - mean) / jnp.sqrt(var + EPS) * g_ref[...] + be_ref[...]
    h = jnp.maximum(h, 0.0)
    h2 = lax.dot_general(
        h, w_ref[...], (((1,), (0,)), ((), ())),
        precision=_HIGHEST, preferred_element_type=jnp.float32,
    )
    out_ref[...] = h2 * dinv


def _tc_post_body(degp_ref, p_ref, hp_ref, b_ref, g_ref, be_ref, batch_ref, wc_ref, bc_ref, out_ref):
    dinv = _dinv_from_degp(degp_ref[...])
    o = (p_ref[0, :N_NODES] + p_ref[1, :N_NODES] + hp_ref[...]) * dinv + b_ref[...]
    mean = jnp.mean(o, axis=0, keepdims=True)
    var = jnp.mean((o - mean) ** 2, axis=0, keepdims=True)
    h = (o - mean) / jnp.sqrt(var + EPS) * g_ref[...] + be_ref[...]
    h = jnp.maximum(h, 0.0)
    gids = lax.broadcasted_iota(jnp.int32, (1, NUM_GRAPHS), 1)
    onehot = (batch_ref[...] == gids).astype(jnp.float32)
    sums = lax.dot_general(
        onehot, h, (((0,), (0,)), ((), ())),
        precision=_HIGHEST, preferred_element_type=jnp.float32,
    )
    counts = jnp.sum(onehot, axis=0)[:, None]
    pooled = sums / jnp.maximum(counts, 1.0)
    out_ref[...] = lax.dot_general(
        pooled, wc_ref[...], (((1,), (0,)), ((), ())),
        precision=_HIGHEST, preferred_element_type=jnp.float32,
    ) + bc_ref[...]


_tc_pre = pl.pallas_call(
    _tc_pre_body, out_shape=jax.ShapeDtypeStruct((N_NODES, HID), jnp.float32)
)
_tc_mid = pl.pallas_call(
    _tc_mid_body, out_shape=jax.ShapeDtypeStruct((N_NODES, HID), jnp.float32)
)
_tc_post = pl.pallas_call(
    _tc_post_body, out_shape=jax.ShapeDtypeStruct((NUM_GRAPHS, OUT_DIM), jnp.float32)
)


@jax.jit
def kernel(x, edge_index, batch, W1, b1, gamma1, beta1, W2, b2, gamma2, beta2, Wc, bc):
    src = edge_index[0].reshape(NW * CHUNKS, C)
    dst = edge_index[1].reshape(NW * CHUNKS, C)
    degp = _sc_deg_kernel()(dst)
    h1p = _tc_pre(x, W1, degp)
    p1 = _sc_conv_kernel()(h1p, src, dst)
    h2p = _tc_mid(degp, p1, h1p, b1[None, :], gamma1[None, :], beta1[None, :], W2)
    p2 = _sc_conv_kernel()(h2p, src, dst)
    return _tc_post(
        degp, p2, h2p, b2[None, :], gamma2[None, :], beta2[None, :],
        batch[:, None], Wc, bc[None, :],
    )
